# baseline (speedup 1.0000x reference)
import functools

import jax
import jax.numpy as jnp
from jax import lax
from jax.experimental import pallas as pl
from jax.experimental.pallas import tpu as pltpu
from jax.experimental.pallas import tpu_sc as plsc

B = 16384
D = 32
NC = 2
NS = 16
L = 16
NW = NC * NS
BPW = B // NW


def _sc_scores(user, pos, neg, users_int, users_pop, items_int, items_pop):
    mesh = plsc.VectorSubcoreMesh(core_axis_name="c", subcore_axis_name="s")

    @functools.partial(
        pl.kernel,
        out_type=[jax.ShapeDtypeStruct((B,), jnp.float32)] * 4,
        mesh=mesh,
        scratch_types=[
            pltpu.VMEM((BPW,), jnp.int32),
            pltpu.VMEM((BPW,), jnp.int32),
            pltpu.VMEM((BPW,), jnp.int32),
            pltpu.VMEM((BPW, D), jnp.float32),
            pltpu.VMEM((BPW, D), jnp.float32),
            pltpu.VMEM((BPW, D), jnp.float32),
            pltpu.VMEM((BPW, D), jnp.float32),
            pltpu.VMEM((BPW, D), jnp.float32),
            pltpu.VMEM((BPW, D), jnp.float32),
            pltpu.VMEM((BPW,), jnp.float32),
            pltpu.VMEM((BPW,), jnp.float32),
            pltpu.VMEM((BPW,), jnp.float32),
            pltpu.VMEM((BPW,), jnp.float32),
            pltpu.SemaphoreType.DMA,
        ],
        compiler_params=pltpu.CompilerParams(
            needs_layout_passes=False, use_tc_tiling_on_sc=False),
    )
    def body(user_h, pos_h, neg_h, ui_h, up_h, ii_h, ip_h,
             o_pint, o_nint, o_ppop, o_npop,
             uidx, pidx, nidx, ui_r, up_r, pi_r, pp_r, ni_r, np_r,
             s_pint, s_nint, s_ppop, s_npop, sem):
        wid = lax.axis_index("s") * NC + lax.axis_index("c")
        base = wid * BPW

        pltpu.sync_copy(user_h.at[pl.ds(base, BPW)], uidx)
        pltpu.sync_copy(pos_h.at[pl.ds(base, BPW)], pidx)
        pltpu.sync_copy(neg_h.at[pl.ds(base, BPW)], nidx)

        c0 = pltpu.async_copy(ui_h.at[uidx], ui_r, sem)
        c1 = pltpu.async_copy(up_h.at[uidx], up_r, sem)
        c2 = pltpu.async_copy(ii_h.at[pidx], pi_r, sem)
        c3 = pltpu.async_copy(ip_h.at[pidx], pp_r, sem)
        c4 = pltpu.async_copy(ii_h.at[nidx], ni_r, sem)
        c5 = pltpu.async_copy(ip_h.at[nidx], np_r, sem)
        c0.wait(); c1.wait(); c2.wait(); c3.wait(); c4.wait(); c5.wait()

        lane = lax.iota(jnp.int32, L)

        def blk_body(blk, _):
            row = lane + blk * L
            zero = jnp.zeros((L,), jnp.float32)
            a_pint, a_nint, a_ppop, a_npop = zero, zero, zero, zero
            for d in range(D):
                col = jnp.full((L,), d, jnp.int32)
                ui = plsc.load_gather(ui_r, [row, col])
                up = plsc.load_gather(up_r, [row, col])
                pi = plsc.load_gather(pi_r, [row, col])
                pp = plsc.load_gather(pp_r, [row, col])
                ni = plsc.load_gather(ni_r, [row, col])
                np_ = plsc.load_gather(np_r, [row, col])
                a_pint = a_pint + ui * pi
                a_nint = a_nint + ui * ni
                a_ppop = a_ppop + up * pp
                a_npop = a_npop + up * np_
            off = blk * L
            s_pint[pl.ds(off, L)] = a_pint
            s_nint[pl.ds(off, L)] = a_nint
            s_ppop[pl.ds(off, L)] = a_ppop
            s_npop[pl.ds(off, L)] = a_npop
            return _

        lax.fori_loop(0, BPW // L, blk_body, None)

        pltpu.sync_copy(s_pint, o_pint.at[pl.ds(base, BPW)])
        pltpu.sync_copy(s_nint, o_nint.at[pl.ds(base, BPW)])
        pltpu.sync_copy(s_ppop, o_ppop.at[pl.ds(base, BPW)])
        pltpu.sync_copy(s_npop, o_npop.at[pl.ds(base, BPW)])

    return body(user, pos, neg, users_int, users_pop, items_int, items_pop)


def _tc_loss_body(pint_ref, nint_ref, ppop_ref, npop_ref, mask_ref, out_ref):
    m = jnp.clip(mask_ref[...], 0.0, 1.0)

    def bpr(x):
        sig = 1.0 / (1.0 + jnp.exp(-x))
        return -jnp.log(sig + 1e-08)

    pint = pint_ref[...]
    nint = nint_ref[...]
    ppop = ppop_ref[...]
    npop = npop_ref[...]
    total = (
        jnp.sum(bpr(pint - nint) * m)
        + jnp.sum(bpr(npop - ppop) * (1.0 - m))
        + jnp.sum(bpr(ppop - npop) * m)
    )
    out_ref[0, 0] = total / B


def kernel(user, pos, neg, mask, pos_period, neg_period,
           users_int, users_pop, items_int, items_pop):
    del pos_period, neg_period
    pint, nint, ppop, npop = _sc_scores(
        user.astype(jnp.int32), pos.astype(jnp.int32), neg.astype(jnp.int32),
        users_int, users_pop, items_int, items_pop)

    shape2d = (B // 128, 128)
    loss = pl.pallas_call(
        _tc_loss_body,
        out_shape=jax.ShapeDtypeStruct((1, 1), jnp.float32),
        out_specs=pl.BlockSpec(memory_space=pltpu.SMEM),
    )(pint.reshape(shape2d), nint.reshape(shape2d),
      ppop.reshape(shape2d), npop.reshape(shape2d),
      mask.astype(jnp.float32).reshape(shape2d))
    return loss[0, 0]
